# baseline (speedup 1.0000x reference)
import jax
import jax.numpy as jnp
from jax import lax
from jax.experimental import pallas as pl
from jax.experimental.pallas import tpu as pltpu
from jax.experimental.pallas import tpu_sc as plsc

H = 50
W = 50
C = 128
P = 7
NB = P * P
CPG = 64
LROWS = 33
UBASE = 25
FSZ = LROWS * W * CPG
NRG = 8
NPAD = 1024
RPW = NPAD // NRG
OSZ = NB * CPG
RB = 4
SPATIAL_SCALE = 0.0625


def _pool_body(feat_hbm, bnd_hbm, out_hbm, feat_v, bnd_v, out_v):
    cid = lax.axis_index("c")
    sid = lax.axis_index("s")
    wid = sid * 2 + cid
    hf = wid % 2
    cg = (wid // 2) % 2
    rg = wid // 4
    pltpu.sync_copy(feat_hbm.at[cg * 2 + hf], feat_v)
    pltpu.sync_copy(bnd_hbm.at[pl.ds(rg * RPW * 32, RPW * 32)], bnd_v)
    ybase = hf * UBASE
    ytop = 33 + hf * 17
    neg = jnp.full((16,), -jnp.inf, dtype=jnp.float32)

    def batch_body(ro, carry):
        for k in range(RB):
            r = ro * RB + k
            row_x = bnd_v[pl.ds(r * 32, 16)]
            row_y = bnd_v[pl.ds(r * 32 + 16, 16)]
            for b in range(NB):
                i, j = b // P, b % P
                xs = row_x[j]
                xe = row_x[8 + j]
                ys = jnp.maximum(row_y[i], ybase)
                ye = jnp.minimum(row_y[8 + i], ytop)

                def y_body(y, acc):
                    base = (y - ybase) * (W * CPG)

                    def x_body(x, acc):
                        a0, a1, a2, a3 = acc
                        px = base + x * CPG
                        a0 = jnp.maximum(a0, feat_v[pl.ds(px, 16)])
                        a1 = jnp.maximum(a1, feat_v[pl.ds(px + 16, 16)])
                        a2 = jnp.maximum(a2, feat_v[pl.ds(px + 32, 16)])
                        a3 = jnp.maximum(a3, feat_v[pl.ds(px + 48, 16)])
                        return (a0, a1, a2, a3)

                    return lax.fori_loop(xs, xe, x_body, acc)

                a0, a1, a2, a3 = lax.fori_loop(ys, ye, y_body, (neg, neg, neg, neg))
                o = k * OSZ + b * CPG
                out_v[pl.ds(o, 16)] = a0
                out_v[pl.ds(o + 16, 16)] = a1
                out_v[pl.ds(o + 32, 16)] = a2
                out_v[pl.ds(o + 48, 16)] = a3
        off = ((hf * 2 + cg) * NPAD + rg * RPW + ro * RB) * OSZ
        pltpu.sync_copy(out_v, out_hbm.at[pl.ds(off, RB * OSZ)])
        return carry

    lax.fori_loop(0, RPW // RB, batch_body, 0)


def kernel(input, rois):
    n = rois.shape[0]
    feat_hw = jnp.transpose(input[0], (1, 2, 0))
    lo = jnp.transpose(feat_hw[:LROWS].reshape(LROWS * W, 2, CPG), (1, 0, 2)).reshape(2, FSZ)
    up = jnp.transpose(feat_hw[UBASE:].reshape((H - UBASE) * W, 2, CPG), (1, 0, 2)).reshape(2, -1)
    up = jnp.pad(up, ((0, 0), (0, FSZ - up.shape[1])))
    feat = jnp.stack([lo[0], up[0], lo[1], up[1]])

    coords = jnp.round(rois[:, 1:] * SPATIAL_SCALE)
    x1 = coords[:, 0]
    y1 = coords[:, 1]
    x2 = coords[:, 2]
    y2 = coords[:, 3]
    roi_w = jnp.clip(x2 - x1 + 1.0, 1.0, None)
    roi_h = jnp.clip(y2 - y1 + 1.0, 1.0, None)
    bin_w = roi_w / P
    bin_h = roi_h / P
    g = jnp.arange(P, dtype=jnp.float32)
    xs = jnp.clip(jnp.floor(g[None, :] * bin_w[:, None]) + x1[:, None], 0.0, float(W)).astype(jnp.int32)
    xe = jnp.clip(jnp.ceil((g[None, :] + 1.0) * bin_w[:, None]) + x1[:, None], 0.0, float(W)).astype(jnp.int32)
    ys = jnp.clip(jnp.floor(g[None, :] * bin_h[:, None]) + y1[:, None], 0.0, float(H)).astype(jnp.int32)
    ye = jnp.clip(jnp.ceil((g[None, :] + 1.0) * bin_h[:, None]) + y1[:, None], 0.0, float(H)).astype(jnp.int32)
    pad1 = lambda a: jnp.pad(a, ((0, 0), (0, 1)))
    bnd = jnp.concatenate([pad1(xs), pad1(xe), pad1(ys), pad1(ye)], axis=1)
    bnd = jnp.pad(bnd, ((0, NPAD - n), (0, 0)))
    bnd = bnd.reshape(NPAD * 32)

    mesh = plsc.VectorSubcoreMesh(core_axis_name="c", subcore_axis_name="s")
    run = pl.kernel(
        _pool_body,
        mesh=mesh,
        out_type=jax.ShapeDtypeStruct((4 * NPAD * OSZ,), jnp.float32),
        scratch_types=[
            pltpu.VMEM((FSZ,), jnp.float32),
            pltpu.VMEM((RPW * 32,), jnp.int32),
            pltpu.VMEM((RB * OSZ,), jnp.float32),
        ],
    )
    out = run(feat, bnd).reshape(2, 2, NPAD, OSZ)
    out = jnp.maximum(out[0], out[1])
    out = jnp.where(jnp.isinf(out), 0.0, out)
    out = out.reshape(2, NPAD, NB, CPG)
    out = jnp.transpose(out, (1, 0, 3, 2)).reshape(NPAD, C, NB)[:n]
    return out.reshape(n, C, P, P)


# speedup vs baseline: 1.4241x; 1.4241x over previous
import jax
import jax.numpy as jnp
from jax import lax
from jax.experimental import pallas as pl
from jax.experimental.pallas import tpu as pltpu
from jax.experimental.pallas import tpu_sc as plsc

H = 50
W = 50
C = 128
P = 7
NB = P * P
XH = W // 2
LROWS = 33
UBASE = 25
FBR = 1664
NRG = 16
NPAD = 1024
RPW = NPAD // NRG
OBR = 112
SPATIAL_SCALE = 0.0625


def _pool_body(feat_hbm, bnd_hbm, par_hbm, out_hbm, feat_v, bnd_v, par_v, out_v):
    cid = lax.axis_index("c")
    sid = lax.axis_index("s")
    wid = sid * 2 + cid
    hf = wid % 2
    rg = wid // 2
    pltpu.sync_copy(feat_hbm.at[pl.ds(pl.multiple_of(hf * FBR, FBR), FBR), :], feat_v)
    pltpu.sync_copy(bnd_hbm.at[pl.ds(rg * RPW * 32, RPW * 32)], bnd_v)
    pltpu.sync_copy(par_hbm, par_v)
    ybase = hf * UBASE
    ytop = 33 + hf * 17
    neg = jnp.full((2, 16), -jnp.inf, dtype=jnp.bfloat16)

    def roi_body(r, carry):
        row_x = bnd_v[pl.ds(r * 32, 16)]
        row_y = bnd_v[pl.ds(r * 32 + 16, 16)]
        for b in range(NB):
            i, j = b // P, b % P
            xs = row_x[j]
            xe = row_x[8 + j]
            ys = jnp.maximum(row_y[i], ybase)
            ye = jnp.minimum(row_y[8 + i], ytop)
            xh0 = xs // 2
            xh1 = (xe + 1) // 2

            def y_body(y, acc):
                rowp = (y - ybase) * XH

                def x_body(xh, acc):
                    rr = pl.multiple_of((rowp + xh) * 2, 2)
                    x0 = xh * 2
                    s0 = ((x0 >= xs) & (x0 < xe)).astype(jnp.int32)
                    s1 = ((x0 + 1 >= xs) & (x0 + 1 < xe)).astype(jnp.int32)
                    sel = pl.multiple_of((3 - 2 * s0 - s1) * 2, 2)
                    mb = par_v[pl.ds(sel, 2), pl.ds(0, 16)]
                    return tuple(
                        jnp.maximum(a, jnp.minimum(mb, feat_v[pl.ds(rr, 2), pl.ds(16 * k, 16)]))
                        for k, a in enumerate(acc)
                    )

                return lax.fori_loop(xh0, xh1, x_body, acc)

            accs = lax.fori_loop(ys, ye, y_body, (neg,) * 8)
            for k in range(8):
                out_v[pl.ds(2 * b, 2), pl.ds(16 * k, 16)] = accs[k]
        off = pl.multiple_of((hf * NPAD + rg * RPW + r) * OBR, OBR)
        pltpu.sync_copy(out_v, out_hbm.at[pl.ds(off, OBR), :])
        return carry

    lax.fori_loop(0, RPW, roi_body, 0)


def kernel(input, rois):
    n = rois.shape[0]
    feat_hw = jnp.transpose(input[0], (1, 2, 0)).astype(jnp.bfloat16)
    fp = feat_hw.reshape(H * XH * 2, C)
    lo = jnp.pad(fp[: LROWS * XH * 2], ((0, FBR - LROWS * XH * 2), (0, 0)))
    up = jnp.pad(fp[UBASE * XH * 2:], ((0, FBR - (H - UBASE) * XH * 2), (0, 0)))
    feat = jnp.concatenate([lo, up])
    valid = jnp.array([1, 1, 1, 0, 0, 1, 0, 0], jnp.float32)
    par = jnp.broadcast_to(
        jnp.where(valid > 0, jnp.inf, -jnp.inf)[:, None].astype(jnp.bfloat16), (8, C)
    )

    coords = jnp.round(rois[:, 1:] * SPATIAL_SCALE)
    x1 = coords[:, 0]
    y1 = coords[:, 1]
    x2 = coords[:, 2]
    y2 = coords[:, 3]
    roi_w = jnp.clip(x2 - x1 + 1.0, 1.0, None)
    roi_h = jnp.clip(y2 - y1 + 1.0, 1.0, None)
    bin_w = roi_w / P
    bin_h = roi_h / P
    g = jnp.arange(P, dtype=jnp.float32)
    xs = jnp.clip(jnp.floor(g[None, :] * bin_w[:, None]) + x1[:, None], 0.0, float(W)).astype(jnp.int32)
    xe = jnp.clip(jnp.ceil((g[None, :] + 1.0) * bin_w[:, None]) + x1[:, None], 0.0, float(W)).astype(jnp.int32)
    ys = jnp.clip(jnp.floor(g[None, :] * bin_h[:, None]) + y1[:, None], 0.0, float(H)).astype(jnp.int32)
    ye = jnp.clip(jnp.ceil((g[None, :] + 1.0) * bin_h[:, None]) + y1[:, None], 0.0, float(H)).astype(jnp.int32)
    pad1 = lambda a: jnp.pad(a, ((0, 0), (0, 1)))
    bnd = jnp.concatenate([pad1(xs), pad1(xe), pad1(ys), pad1(ye)], axis=1)
    bnd = jnp.pad(bnd, ((0, NPAD - n), (0, 0)))
    bnd = bnd.reshape(NPAD * 32)

    mesh = plsc.VectorSubcoreMesh(core_axis_name="c", subcore_axis_name="s")
    run = pl.kernel(
        _pool_body,
        mesh=mesh,
        out_type=jax.ShapeDtypeStruct((2 * NPAD * OBR, C), jnp.bfloat16),
        scratch_types=[
            pltpu.VMEM((2 * FBR // 2, C), jnp.bfloat16),
            pltpu.VMEM((RPW * 32,), jnp.int32),
            pltpu.VMEM((8, C), jnp.bfloat16),
            pltpu.VMEM((OBR, C), jnp.bfloat16),
        ],
    )
    out = run(feat, bnd, par)
    out = out.reshape(2, NPAD, OBR, C)[:, :, : 2 * NB]
    out = out.reshape(2, NPAD, NB, 2, C)
    out = jnp.max(out, axis=(0, 3)).astype(jnp.float32)
    out = jnp.where(jnp.isinf(out), 0.0, out)
    out = jnp.transpose(out, (0, 2, 1))[:n]
    return out.reshape(n, C, P, P)
